# baseline (device time: 22500 ns/iter reference)
import jax
import jax.numpy as jnp
from jax import lax
from jax.experimental import pallas as pl
from jax.experimental.pallas import tpu as pltpu

T = 512
D = 1024
V_LOCAL = 8192
V_NODE = 4096
CH = 1024
N_CH = V_NODE // CH
NBUF = 4


def kernel(x, W, labels):
    def body(x_ref, w_ref, labels_ref, out_ref,
             w_buf, send_buf, recv_buf, load_sems, send_sem, recv_sem):
        my_x = lax.axis_index("x")
        my_y = lax.axis_index("y")
        my_id = my_x * 2 + my_y
        peers = [
            (my_x, 1 - my_y),
            (1 - my_x, my_y),
            (1 - my_x, 1 - my_y),
        ]
        col0 = my_x * V_NODE

        barrier = pltpu.get_barrier_semaphore()
        for p in peers:
            pl.semaphore_signal(
                barrier, inc=1,
                device_id=p, device_id_type=pl.DeviceIdType.MESH,
            )

        def dma(i):
            return pltpu.make_async_copy(
                w_ref.at[:, pl.ds(col0 + i * CH, CH)],
                w_buf.at[i % NBUF],
                load_sems.at[i % NBUF],
            )

        for i in range(NBUF):
            dma(i).start()

        xv = x_ref[...]
        labels_col = labels_ref[...]
        base = my_y * V_LOCAL + col0
        cols0 = lax.broadcasted_iota(jnp.int32, (T, CH), 1)
        acc = jnp.zeros((T, CH), jnp.float32)
        lacc = jnp.zeros((T, CH), jnp.float32)
        for i in range(N_CH):
            dma(i).wait()
            logits = jnp.dot(xv, w_buf[i % NBUF],
                             preferred_element_type=jnp.float32)
            e = jnp.exp(logits)
            acc += e
            lacc += jnp.where(cols0 == labels_col - (base + i * CH), e, 0.0)
            if i + NBUF < N_CH:
                dma(i + NBUF).start()
        s_acc = jnp.sum(acc, axis=1, keepdims=True)
        l_acc = jnp.sum(lacc, axis=1, keepdims=True)

        send_buf[:, 0:1] = s_acc
        send_buf[:, 1:2] = l_acc

        pl.semaphore_wait(barrier, 3)

        for p in peers:
            pltpu.make_async_remote_copy(
                src_ref=send_buf,
                dst_ref=recv_buf.at[my_id],
                send_sem=send_sem,
                recv_sem=recv_sem,
                device_id=p,
                device_id_type=pl.DeviceIdType.MESH,
            ).start()

        wall = pltpu.make_async_remote_copy(
            src_ref=recv_buf.at[pl.ds(0, 3)],
            dst_ref=recv_buf.at[pl.ds(0, 3)],
            send_sem=send_sem,
            recv_sem=recv_sem,
            device_id=(my_x, my_y),
            device_id_type=pl.DeviceIdType.MESH,
        )
        wall.wait_recv()

        s_tot = s_acc
        l_tot = l_acc
        for p in peers:
            pid = p[0] * 2 + p[1]
            s_tot += recv_buf[pid, :, 0:1]
            l_tot += recv_buf[pid, :, 1:2]
        wall.wait_send()

        out_ref[...] = jnp.log(s_tot) - jnp.log(l_tot)

    out = pl.pallas_call(
        body,
        out_shape=jax.ShapeDtypeStruct((T, 1), jnp.float32),
        in_specs=[
            pl.BlockSpec(memory_space=pltpu.MemorySpace.VMEM),
            pl.BlockSpec(memory_space=pltpu.MemorySpace.HBM),
            pl.BlockSpec(memory_space=pltpu.MemorySpace.VMEM),
        ],
        out_specs=pl.BlockSpec(memory_space=pltpu.MemorySpace.VMEM),
        scratch_shapes=[
            pltpu.VMEM((NBUF, D, CH), jnp.float32),
            pltpu.VMEM((T, 8), jnp.float32),
            pltpu.VMEM((4, T, 8), jnp.float32),
            pltpu.SemaphoreType.DMA((NBUF,)),
            pltpu.SemaphoreType.DMA,
            pltpu.SemaphoreType.DMA,
        ],
        compiler_params=pltpu.CompilerParams(collective_id=0),
    )(x, W, labels.reshape(T, 1).astype(jnp.int32))
    return out.reshape(T)
